# baseline (speedup 1.0000x reference)
import functools

import jax
import jax.numpy as jnp
from jax import lax
from jax.experimental import pallas as pl
from jax.experimental.pallas import tpu as pltpu
from jax.experimental.pallas import tpu_sc as plsc

N_NODES = 10000
D_IN = 128
D_HID = 64

NC = 2
NS = 16
NW = NC * NS

CHUNK = 128
NP = 10112
RPT = NP // NS

_mesh = plsc.VectorSubcoreMesh(
    core_axis_name="c", subcore_axis_name="s", num_cores=NC, num_subcores=NS)


NPD = 10240
DRT = NPD // NW


def _make_deg_count_kernel(K):
  @functools.partial(
      pl.kernel,
      out_type=jax.ShapeDtypeStruct((NW, NPD), jnp.float32),
      mesh=_mesh,
      scratch_types=[
          pltpu.VMEM((K, CHUNK), jnp.int32),
          pltpu.VMEM((NPD,), jnp.float32),
      ],
      compiler_params=pltpu.CompilerParams(use_tc_tiling_on_sc=False, needs_layout_passes=False, internal_scratch_in_bytes=1048576),
  )
  def deg_count(dst2d, out, idx_v, deg_v):
    cid = lax.axis_index("c")
    sid = lax.axis_index("s")
    wid = cid * NS + sid
    pltpu.sync_copy(dst2d.at[wid], idx_v)
    zeros = jnp.zeros((16,), jnp.float32)

    @pl.loop(0, NPD // 16)
    def _(i):
      deg_v[pl.ds(i * 16, 16)] = zeros

    ones = jnp.ones((16,), jnp.float32)

    @pl.loop(0, K)
    def _(j):
      row = idx_v.at[j]
      for i in range(CHUNK // 16):
        idx16 = row[pl.ds(i * 16, 16)]
        plsc.addupdate_scatter(deg_v, [idx16], ones)

    pltpu.sync_copy(deg_v, out.at[wid])

  return deg_count


@functools.partial(
    pl.kernel,
    out_type=jax.ShapeDtypeStruct((NPD,), jnp.float32),
    mesh=_mesh,
    scratch_types=[
        pltpu.VMEM((NW, DRT), jnp.float32),
    ],
    compiler_params=pltpu.CompilerParams(use_tc_tiling_on_sc=False, needs_layout_passes=False, internal_scratch_in_bytes=1048576),
)
def _deg_combine_kernel(partials, out, buf_v):
  cid = lax.axis_index("c")
  sid = lax.axis_index("s")
  wid = cid * NS + sid
  pltpu.sync_copy(partials.at[:, pl.ds(wid * DRT, DRT)], buf_v)
  for v in range(DRT // 16):
    acc = buf_v[0, pl.ds(v * 16, 16)]
    for t in range(1, NW):
      acc = acc + buf_v[t, pl.ds(v * 16, 16)]
    buf_v[0, pl.ds(v * 16, 16)] = acc
  pltpu.sync_copy(buf_v.at[0], out.at[pl.ds(wid * DRT, DRT)])


NBUF = 2


def _make_edge_kernel(K):
  assert K % NBUF == 0 and K >= 2 * NBUF

  @functools.partial(
      pl.kernel,
      out_type=jax.ShapeDtypeStruct((NC, NP, D_HID), jnp.float32),
      mesh=_mesh,
      scratch_types=[
          pltpu.VMEM((K, CHUNK), jnp.int32),
          pltpu.VMEM((K, CHUNK), jnp.int32),
          pltpu.VMEM((NBUF, CHUNK, D_HID), jnp.float32),
          pltpu.VMEM((RPT, D_HID), jnp.float32),
          pltpu.VMEM_SHARED((NP, D_HID), jnp.float32),
      ] + [pltpu.SemaphoreType.DMA] * NBUF,
      compiler_params=pltpu.CompilerParams(use_tc_tiling_on_sc=False, needs_layout_passes=False),
  )
  def edge_kernel(table, src2d, dst2d, zeros64, out, src_v, dst_v, bufs,
                  stage_v, agg_sh, *gsem):
    cid = lax.axis_index("c")
    sid = lax.axis_index("s")
    wid = cid * NS + sid
    pltpu.sync_copy(zeros64, stage_v)
    pltpu.sync_copy(stage_v, agg_sh.at[pl.ds(sid * RPT, RPT)])
    pltpu.sync_copy(src2d.at[wid], src_v)
    pltpu.sync_copy(dst2d.at[wid], dst_v)
    plsc.subcore_barrier()

    for b in range(NBUF):
      pltpu.async_copy(table.at[src_v.at[b]], bufs.at[b], gsem[b])

    @pl.loop(0, K // NBUF)
    def _(g):
      base = g * NBUF
      for b in range(NBUF):
        j = base + b
        pltpu.make_async_copy(
            table.at[src_v.at[j]], bufs.at[b], gsem[b]).wait()
        pltpu.sync_copy(bufs.at[b], agg_sh.at[dst_v.at[j]], add=True)

        @pl.when(j + NBUF < K)
        def _():
          pltpu.async_copy(
              table.at[src_v.at[j + NBUF]], bufs.at[b], gsem[b])

    plsc.subcore_barrier()
    pltpu.sync_copy(agg_sh.at[pl.ds(sid * RPT, RPT)], stage_v)
    pltpu.sync_copy(stage_v, out.at[cid].at[pl.ds(sid * RPT, RPT)])

  return edge_kernel


BR = 2528


def _tc_layer1(xp, W1, d):
  def body(x_ref, w_ref, d_ref, scaled_ref, dinv_ref):
    dinv = lax.rsqrt(jnp.maximum(d_ref[...], 1.0))
    h = jnp.dot(x_ref[...], w_ref[...], preferred_element_type=jnp.float32)
    scaled_ref[...] = h * dinv
    dinv_ref[...] = dinv

  grid = (NP // BR,)
  return pl.pallas_call(
      body,
      grid=grid,
      in_specs=[
          pl.BlockSpec((BR, D_IN), lambda i: (i, 0)),
          pl.BlockSpec((D_IN, D_HID), lambda i: (0, 0)),
          pl.BlockSpec((BR, 1), lambda i: (i, 0)),
      ],
      out_specs=[
          pl.BlockSpec((BR, D_HID), lambda i: (i, 0)),
          pl.BlockSpec((BR, 1), lambda i: (i, 0)),
      ],
      out_shape=[
          jax.ShapeDtypeStruct((NP, D_HID), jnp.float32),
          jax.ShapeDtypeStruct((NP, 1), jnp.float32),
      ],
  )(xp, W1, d)


def _tc_layer2(a0, a1, dinv, W2, b1):
  def body(a0_ref, a1_ref, dinv_ref, w_ref, b_ref, out_ref):
    dinv = dinv_ref[...]
    h1 = jnp.maximum((a0_ref[...] + a1_ref[...]) * dinv + b_ref[...], 0.0)
    out_ref[...] = jnp.dot(
        h1, w_ref[...], preferred_element_type=jnp.float32) * dinv

  grid = (NP // BR,)
  return pl.pallas_call(
      body,
      grid=grid,
      in_specs=[
          pl.BlockSpec((BR, D_HID), lambda i: (i, 0)),
          pl.BlockSpec((BR, D_HID), lambda i: (i, 0)),
          pl.BlockSpec((BR, 1), lambda i: (i, 0)),
          pl.BlockSpec((D_HID, D_HID), lambda i: (0, 0)),
          pl.BlockSpec((1, D_HID), lambda i: (0, 0)),
      ],
      out_specs=pl.BlockSpec((BR, D_HID), lambda i: (i, 0)),
      out_shape=jax.ShapeDtypeStruct((NP, D_HID), jnp.float32),
  )(a0, a1, dinv, W2, b1)


def _tc_heads(a0, a1, dinv, b2, W_he, b_he):
  def body(a0_ref, a1_ref, dinv_ref, b2_ref, w_ref, bh_ref, out_ref):
    dinv = dinv_ref[...]
    h = jnp.maximum((a0_ref[...] + a1_ref[...]) * dinv + b2_ref[...], 0.0)
    out_ref[...] = jnp.dot(
        h, w_ref[...], preferred_element_type=jnp.float32) + bh_ref[...]

  grid = (NP // BR,)
  return pl.pallas_call(
      body,
      grid=grid,
      in_specs=[
          pl.BlockSpec((BR, D_HID), lambda i: (i, 0)),
          pl.BlockSpec((BR, D_HID), lambda i: (i, 0)),
          pl.BlockSpec((BR, 1), lambda i: (i, 0)),
          pl.BlockSpec((1, D_HID), lambda i: (0, 0)),
          pl.BlockSpec((D_HID, 2), lambda i: (0, 0)),
          pl.BlockSpec((1, 2), lambda i: (0, 0)),
      ],
      out_specs=pl.BlockSpec((BR, 2), lambda i: (i, 0)),
      out_shape=jax.ShapeDtypeStruct((NP, 2), jnp.float32),
  )(a0, a1, dinv, b2, W_he, b_he)


def kernel(x, edge_index, W1, b1, W2, b2, W_time, b_time, W_event, b_event):
  n_edges = edge_index.shape[1]
  e_tot = n_edges + N_NODES
  K = -(-e_tot // (NW * CHUNK))
  K = -(-K // NBUF) * NBUF
  ep = K * NW * CHUNK

  src = edge_index[0]
  dst = edge_index[1]
  self_ix = jnp.arange(N_NODES, dtype=jnp.int32)
  pad = ep - e_tot
  srcp = jnp.concatenate(
      [src, self_ix, jnp.zeros((pad,), jnp.int32)]).reshape(NW, K, CHUNK)
  dstp = jnp.concatenate(
      [dst, self_ix,
       jnp.full((pad,), N_NODES, jnp.int32)]).reshape(NW, K, CHUNK)
  xp = jnp.concatenate(
      [x, jnp.zeros((NP - N_NODES, D_IN), jnp.float32)])

  zeros64 = jnp.zeros((RPT, D_HID), jnp.float32)

  degp = _make_deg_count_kernel(K)(dstp)
  deg = _deg_combine_kernel(degp)
  d = deg[:NP].reshape(NP, 1)

  scaled1, dinv = _tc_layer1(xp, W1, d)

  edge_k = _make_edge_kernel(K)
  agg1 = edge_k(scaled1, srcp, dstp, zeros64)
  scaled2 = _tc_layer2(agg1[0], agg1[1], dinv, W2, b1.reshape(1, D_HID))

  agg2 = edge_k(scaled2, srcp, dstp, zeros64)
  W_he = jnp.concatenate([W_time, W_event], axis=1)
  b_he = jnp.concatenate([b_time, b_event]).reshape(1, 2)
  out = _tc_heads(agg2[0], agg2[1], dinv, b2.reshape(1, D_HID), W_he, b_he)
  return (out[:N_NODES, :1], out[:N_NODES, 1:2])


# speedup vs baseline: 1.0096x; 1.0096x over previous
import functools

import jax
import jax.numpy as jnp
from jax import lax
from jax.experimental import pallas as pl
from jax.experimental.pallas import tpu as pltpu
from jax.experimental.pallas import tpu_sc as plsc

N_NODES = 10000
D_IN = 128
D_HID = 64

NC = 2
NS = 16
NW = NC * NS

CHUNK = 128
NP = 10112
RPT = NP // NS

_mesh = plsc.VectorSubcoreMesh(
    core_axis_name="c", subcore_axis_name="s", num_cores=NC, num_subcores=NS)


NPD = 10240
DRT = NPD // NW


def _make_deg_count_kernel(K):
  @functools.partial(
      pl.kernel,
      out_type=jax.ShapeDtypeStruct((NW, NPD), jnp.float32),
      mesh=_mesh,
      scratch_types=[
          pltpu.VMEM((K, CHUNK), jnp.int32),
          pltpu.VMEM((NPD,), jnp.float32),
      ],
      compiler_params=pltpu.CompilerParams(use_tc_tiling_on_sc=False, needs_layout_passes=False, internal_scratch_in_bytes=1048576),
  )
  def deg_count(dst2d, out, idx_v, deg_v):
    cid = lax.axis_index("c")
    sid = lax.axis_index("s")
    wid = cid * NS + sid
    pltpu.sync_copy(dst2d.at[wid], idx_v)
    zeros = jnp.zeros((16,), jnp.float32)

    @pl.loop(0, NPD // 16)
    def _(i):
      deg_v[pl.ds(i * 16, 16)] = zeros

    ones = jnp.ones((16,), jnp.float32)

    @pl.loop(0, K)
    def _(j):
      row = idx_v.at[j]
      for i in range(CHUNK // 16):
        idx16 = row[pl.ds(i * 16, 16)]
        plsc.addupdate_scatter(deg_v, [idx16], ones)

    pltpu.sync_copy(deg_v, out.at[wid])

  return deg_count


@functools.partial(
    pl.kernel,
    out_type=jax.ShapeDtypeStruct((NPD,), jnp.float32),
    mesh=_mesh,
    scratch_types=[
        pltpu.VMEM((NW, DRT), jnp.float32),
    ],
    compiler_params=pltpu.CompilerParams(use_tc_tiling_on_sc=False, needs_layout_passes=False, internal_scratch_in_bytes=1048576),
)
def _deg_combine_kernel(partials, out, buf_v):
  cid = lax.axis_index("c")
  sid = lax.axis_index("s")
  wid = cid * NS + sid
  pltpu.sync_copy(partials.at[:, pl.ds(wid * DRT, DRT)], buf_v)
  for v in range(DRT // 16):
    acc = buf_v[0, pl.ds(v * 16, 16)]
    for t in range(1, NW):
      acc = acc + buf_v[t, pl.ds(v * 16, 16)]
    buf_v[0, pl.ds(v * 16, 16)] = acc
  pltpu.sync_copy(buf_v.at[0], out.at[pl.ds(wid * DRT, DRT)])


NBUF = 2


def _make_edge_kernel(K):
  assert K % NBUF == 0 and K >= 2 * NBUF

  @functools.partial(
      pl.kernel,
      out_type=jax.ShapeDtypeStruct((NC, NP, D_HID), jnp.float32),
      mesh=_mesh,
      scratch_types=[
          pltpu.VMEM((K, CHUNK), jnp.int32),
          pltpu.VMEM((K, CHUNK), jnp.int32),
          pltpu.VMEM((NBUF, CHUNK, D_HID), jnp.float32),
          pltpu.VMEM((RPT, D_HID), jnp.float32),
          pltpu.VMEM_SHARED((NP, D_HID), jnp.float32),
      ] + [pltpu.SemaphoreType.DMA] * NBUF,
      compiler_params=pltpu.CompilerParams(use_tc_tiling_on_sc=False, needs_layout_passes=False),
  )
  def edge_kernel(table, src2d, dst2d, zeros64, out, src_v, dst_v, bufs,
                  stage_v, agg_sh, *gsem):
    cid = lax.axis_index("c")
    sid = lax.axis_index("s")
    wid = cid * NS + sid
    pltpu.sync_copy(zeros64, stage_v)
    pltpu.sync_copy(stage_v, agg_sh.at[pl.ds(sid * RPT, RPT)])
    pltpu.sync_copy(src2d.at[wid], src_v)
    pltpu.sync_copy(dst2d.at[wid], dst_v)
    plsc.subcore_barrier()

    for b in range(NBUF):
      pltpu.async_copy(table.at[src_v.at[b]], bufs.at[b], gsem[b])

    @pl.loop(0, K // NBUF)
    def _(g):
      base = g * NBUF
      for b in range(NBUF):
        j = base + b
        pltpu.make_async_copy(
            table.at[src_v.at[j]], bufs.at[b], gsem[b]).wait()
        pltpu.sync_copy(bufs.at[b], agg_sh.at[dst_v.at[j]], add=True)

        @pl.when(j + NBUF < K)
        def _():
          pltpu.async_copy(
              table.at[src_v.at[j + NBUF]], bufs.at[b], gsem[b])

    plsc.subcore_barrier()
    pltpu.sync_copy(agg_sh.at[pl.ds(sid * RPT, RPT)], stage_v)
    pltpu.sync_copy(stage_v, out.at[cid].at[pl.ds(sid * RPT, RPT)])

  return edge_kernel


BR = 2528


def _tc_layer1(xp, W1, d):
  def body(x_ref, w_ref, d_ref, scaled_ref, dinv_ref):
    dinv = lax.rsqrt(jnp.maximum(d_ref[...], 1.0))
    h = jnp.dot(x_ref[...], w_ref[...], preferred_element_type=jnp.float32)
    scaled_ref[...] = h * dinv
    dinv_ref[...] = dinv

  grid = (NP // BR,)
  return pl.pallas_call(
      body,
      grid=grid,
      in_specs=[
          pl.BlockSpec((BR, D_IN), lambda i: (i, 0)),
          pl.BlockSpec((D_IN, D_HID), lambda i: (0, 0)),
          pl.BlockSpec((BR, 1), lambda i: (i, 0)),
      ],
      out_specs=[
          pl.BlockSpec((BR, D_HID), lambda i: (i, 0)),
          pl.BlockSpec((BR, 1), lambda i: (i, 0)),
      ],
      out_shape=[
          jax.ShapeDtypeStruct((NP, D_HID), jnp.float32),
          jax.ShapeDtypeStruct((NP, 1), jnp.float32),
      ],
  )(xp, W1, d)


def _tc_layer2(a0, a1, dinv, W2, b1):
  def body(a0_ref, a1_ref, dinv_ref, w_ref, b_ref, out_ref):
    dinv = dinv_ref[...]
    h1 = jnp.maximum((a0_ref[...] + a1_ref[...]) * dinv + b_ref[...], 0.0)
    out_ref[...] = jnp.dot(
        h1, w_ref[...], preferred_element_type=jnp.float32) * dinv

  grid = (NP // BR,)
  return pl.pallas_call(
      body,
      grid=grid,
      in_specs=[
          pl.BlockSpec((BR, D_HID), lambda i: (i, 0)),
          pl.BlockSpec((BR, D_HID), lambda i: (i, 0)),
          pl.BlockSpec((BR, 1), lambda i: (i, 0)),
          pl.BlockSpec((D_HID, D_HID), lambda i: (0, 0)),
          pl.BlockSpec((1, D_HID), lambda i: (0, 0)),
      ],
      out_specs=pl.BlockSpec((BR, D_HID), lambda i: (i, 0)),
      out_shape=jax.ShapeDtypeStruct((NP, D_HID), jnp.float32),
  )(a0, a1, dinv, W2, b1)


def _tc_heads(a0, a1, dinv, b2, W_he, b_he):
  def body(a0_ref, a1_ref, dinv_ref, b2_ref, w_ref, bh_ref, out_ref):
    dinv = dinv_ref[...]
    h = jnp.maximum((a0_ref[...] + a1_ref[...]) * dinv + b2_ref[...], 0.0)
    out_ref[...] = jnp.dot(
        h, w_ref[...], preferred_element_type=jnp.float32) + bh_ref[...]

  grid = (NP // BR,)
  return pl.pallas_call(
      body,
      grid=grid,
      in_specs=[
          pl.BlockSpec((BR, D_HID), lambda i: (i, 0)),
          pl.BlockSpec((BR, D_HID), lambda i: (i, 0)),
          pl.BlockSpec((BR, 1), lambda i: (i, 0)),
          pl.BlockSpec((1, D_HID), lambda i: (0, 0)),
          pl.BlockSpec((D_HID, 2), lambda i: (0, 0)),
          pl.BlockSpec((1, 2), lambda i: (0, 0)),
      ],
      out_specs=pl.BlockSpec((BR, 2), lambda i: (i, 0)),
      out_shape=jax.ShapeDtypeStruct((NP, 2), jnp.float32),
  )(a0, a1, dinv, b2, W_he, b_he)


def kernel(x, edge_index, W1, b1, W2, b2, W_time, b_time, W_event, b_event):
  n_edges = edge_index.shape[1]
  e_tot = n_edges + N_NODES
  K = -(-e_tot // (NW * CHUNK))
  K = -(-K // NBUF) * NBUF
  ep = K * NW * CHUNK

  src = edge_index[0]
  dst = edge_index[1]
  self_ix = jnp.arange(N_NODES, dtype=jnp.int32)
  pad = ep - e_tot
  srcp = jnp.concatenate(
      [src, self_ix, jnp.zeros((pad,), jnp.int32)]).reshape(NW, K, CHUNK)
  junk = N_NODES + (jnp.arange(pad, dtype=jnp.int32) % (NP - N_NODES))
  dstp = jnp.concatenate(
      [dst, self_ix, junk]).reshape(NW, K, CHUNK)
  xp = jnp.concatenate(
      [x, jnp.zeros((NP - N_NODES, D_IN), jnp.float32)])

  zeros64 = jnp.zeros((RPT, D_HID), jnp.float32)

  degp = _make_deg_count_kernel(K)(dstp)
  deg = _deg_combine_kernel(degp)
  d = deg[:NP].reshape(NP, 1)

  scaled1, dinv = _tc_layer1(xp, W1, d)

  edge_k = _make_edge_kernel(K)
  agg1 = edge_k(scaled1, srcp, dstp, zeros64)
  scaled2 = _tc_layer2(agg1[0], agg1[1], dinv, W2, b1.reshape(1, D_HID))

  agg2 = edge_k(scaled2, srcp, dstp, zeros64)
  W_he = jnp.concatenate([W_time, W_event], axis=1)
  b_he = jnp.concatenate([b_time, b_event]).reshape(1, 2)
  out = _tc_heads(agg2[0], agg2[1], dinv, b2.reshape(1, D_HID), W_he, b_he)
  return (out[:N_NODES, :1], out[:N_NODES, 1:2])


# speedup vs baseline: 2.2345x; 2.2132x over previous
import functools

import jax
import jax.numpy as jnp
from jax import lax
from jax.experimental import pallas as pl
from jax.experimental.pallas import tpu as pltpu
from jax.experimental.pallas import tpu_sc as plsc

N_NODES = 10000
D_IN = 128
D_HID = 64

NC = 2
NS = 16
NW = NC * NS

CHUNK = 128
NP = 10112
RPT = NP // NS
NPD = 10240
DRT = NPD // NW

NBUF = 3

_sc_params = pltpu.CompilerParams(
    use_tc_tiling_on_sc=False, needs_layout_passes=False)
_mesh = plsc.VectorSubcoreMesh(
    core_axis_name="c", subcore_axis_name="s", num_cores=NC, num_subcores=NS)


def _chunk_split(n_chunks):
  base = n_chunks // NW
  left = n_chunks - base * NW
  return base, left


def _make_deg_count_kernel(n_chunks):
  base, left = _chunk_split(n_chunks)

  @functools.partial(
      pl.kernel,
      out_type=jax.ShapeDtypeStruct((NW, NPD), jnp.float32),
      mesh=_mesh,
      scratch_types=[
          pltpu.VMEM((base + 1, CHUNK), jnp.int32),
          pltpu.VMEM((NPD,), jnp.float32),
      ],
      compiler_params=_sc_params,
  )
  def deg_count(edges2d, out, idx_v, deg_v):
    cid = lax.axis_index("c")
    sid = lax.axis_index("s")
    wid = cid * NS + sid
    pltpu.sync_copy(edges2d.at[1].at[pl.ds(wid * base, base)],
                    idx_v.at[pl.ds(0, base)])

    @pl.when(wid < left)
    def _():
      pltpu.sync_copy(edges2d.at[1].at[pl.ds(NW * base + wid, 1)],
                      idx_v.at[pl.ds(base, 1)])

    zeros = jnp.zeros((16,), jnp.float32)

    @pl.loop(0, NPD // 16)
    def _(i):
      deg_v[pl.ds(i * 16, 16)] = zeros

    ones = jnp.ones((16,), jnp.float32)
    nmine = jnp.where(wid < left, base + 1, base)

    @pl.loop(0, nmine)
    def _(j):
      row = idx_v.at[j]
      for i in range(CHUNK // 16):
        idx16 = row[pl.ds(i * 16, 16)]
        plsc.addupdate_scatter(deg_v, [idx16], ones)

    pltpu.sync_copy(deg_v, out.at[wid])

  return deg_count


@functools.partial(
    pl.kernel,
    out_type=jax.ShapeDtypeStruct((NPD,), jnp.float32),
    mesh=_mesh,
    scratch_types=[
        pltpu.VMEM((NW, DRT), jnp.float32),
        pltpu.VMEM((DRT,), jnp.float32),
    ],
    compiler_params=_sc_params,
)
def _deg_combine_kernel(partials, out, buf_v, acc_v):
  cid = lax.axis_index("c")
  sid = lax.axis_index("s")
  wid = cid * NS + sid
  pltpu.sync_copy(partials.at[:, pl.ds(wid * DRT, DRT)], buf_v)
  for v in range(DRT // 16):
    acc = buf_v[0, pl.ds(v * 16, 16)]
    for t in range(1, NW):
      acc = acc + buf_v[t, pl.ds(v * 16, 16)]
    acc_v[pl.ds(v * 16, 16)] = acc
  pltpu.sync_copy(acc_v, out.at[pl.ds(wid * DRT, DRT)])


def _make_edge_kernel(n_chunks):
  base, left = _chunk_split(n_chunks)
  assert base % NBUF == 0 and base >= 2 * NBUF

  @functools.partial(
      pl.kernel,
      out_type=jax.ShapeDtypeStruct((NC, NP, D_HID), jnp.float32),
      mesh=_mesh,
      scratch_types=[
          pltpu.VMEM((base + 1, CHUNK), jnp.int32),
          pltpu.VMEM((base + 1, CHUNK), jnp.int32),
          pltpu.VMEM((NBUF, CHUNK, D_HID), jnp.float32),
          pltpu.VMEM((RPT, D_HID), jnp.float32),
          pltpu.VMEM_SHARED((NP, D_HID), jnp.float32),
      ] + [pltpu.SemaphoreType.DMA] * NBUF,
      compiler_params=_sc_params,
  )
  def edge_kernel(table, edges2d, zeros64, out, src_v, dst_v, bufs,
                  stage_v, agg_sh, *gsem):
    cid = lax.axis_index("c")
    sid = lax.axis_index("s")
    wid = cid * NS + sid
    pltpu.sync_copy(zeros64, stage_v)
    pltpu.sync_copy(stage_v, agg_sh.at[pl.ds(sid * RPT, RPT)])
    pltpu.sync_copy(edges2d.at[0].at[pl.ds(wid * base, base)],
                    src_v.at[pl.ds(0, base)])
    pltpu.sync_copy(edges2d.at[1].at[pl.ds(wid * base, base)],
                    dst_v.at[pl.ds(0, base)])

    @pl.when(wid < left)
    def _():
      pltpu.sync_copy(edges2d.at[0].at[pl.ds(NW * base + wid, 1)],
                      src_v.at[pl.ds(base, 1)])
      pltpu.sync_copy(edges2d.at[1].at[pl.ds(NW * base + wid, 1)],
                      dst_v.at[pl.ds(base, 1)])

    plsc.subcore_barrier()

    for b in range(NBUF):
      pltpu.async_copy(table.at[src_v.at[b]], bufs.at[b], gsem[b])

    @pl.loop(0, base // NBUF)
    def _(g):
      bb = g * NBUF
      for b in range(NBUF):
        j = bb + b
        pltpu.make_async_copy(
            table.at[src_v.at[j]], bufs.at[b], gsem[b]).wait()
        pltpu.sync_copy(bufs.at[b], agg_sh.at[dst_v.at[j]], add=True)

        @pl.when(j + NBUF < base)
        def _():
          pltpu.async_copy(
              table.at[src_v.at[j + NBUF]], bufs.at[b], gsem[b])

    @pl.when(wid < left)
    def _():
      pltpu.async_copy(table.at[src_v.at[base]], bufs.at[0], gsem[0]).wait()
      pltpu.sync_copy(bufs.at[0], agg_sh.at[dst_v.at[base]], add=True)

    plsc.subcore_barrier()
    pltpu.sync_copy(agg_sh.at[pl.ds(sid * RPT, RPT)], stage_v)
    pltpu.sync_copy(stage_v, out.at[cid].at[pl.ds(sid * RPT, RPT)])

  return edge_kernel


BR = 2000


def _tc_layer1(x, W1, d):
  def body(x_ref, w_ref, d_ref, scaled_ref, dinv_ref):
    dinv = lax.rsqrt(d_ref[...] + 1.0)
    h = jnp.dot(x_ref[...], w_ref[...], preferred_element_type=jnp.float32)
    scaled_ref[...] = h * dinv
    dinv_ref[...] = dinv

  grid = (N_NODES // BR,)
  return pl.pallas_call(
      body,
      grid=grid,
      in_specs=[
          pl.BlockSpec((BR, D_IN), lambda i: (i, 0)),
          pl.BlockSpec((D_IN, D_HID), lambda i: (0, 0)),
          pl.BlockSpec((BR, 1), lambda i: (i, 0)),
      ],
      out_specs=[
          pl.BlockSpec((BR, D_HID), lambda i: (i, 0)),
          pl.BlockSpec((BR, 1), lambda i: (i, 0)),
      ],
      out_shape=[
          jax.ShapeDtypeStruct((N_NODES, D_HID), jnp.float32),
          jax.ShapeDtypeStruct((N_NODES, 1), jnp.float32),
      ],
  )(x, W1, d)


def _tc_layer2(agg, scaled1, dinv, W2, b1):
  def body(a0_ref, a1_ref, s_ref, dinv_ref, w_ref, b_ref, out_ref):
    dinv = dinv_ref[...]
    z = (a0_ref[0] + a1_ref[0] + s_ref[...]) * dinv + b_ref[...]
    h1 = jnp.maximum(z, 0.0)
    out_ref[...] = jnp.dot(
        h1, w_ref[...], preferred_element_type=jnp.float32) * dinv

  grid = (N_NODES // BR,)
  return pl.pallas_call(
      body,
      grid=grid,
      in_specs=[
          pl.BlockSpec((1, BR, D_HID), lambda i: (0, i, 0)),
          pl.BlockSpec((1, BR, D_HID), lambda i: (1, i, 0)),
          pl.BlockSpec((BR, D_HID), lambda i: (i, 0)),
          pl.BlockSpec((BR, 1), lambda i: (i, 0)),
          pl.BlockSpec((D_HID, D_HID), lambda i: (0, 0)),
          pl.BlockSpec((1, D_HID), lambda i: (0, 0)),
      ],
      out_specs=pl.BlockSpec((BR, D_HID), lambda i: (i, 0)),
      out_shape=jax.ShapeDtypeStruct((N_NODES, D_HID), jnp.float32),
  )(agg, agg, scaled1, dinv, W2, b1)


def _tc_heads(agg, scaled2, dinv, b2, W_he, b_he):
  def body(a0_ref, a1_ref, s_ref, dinv_ref, b2_ref, w_ref, bh_ref,
           t_ref, e_ref):
    dinv = dinv_ref[...]
    z = (a0_ref[0] + a1_ref[0] + s_ref[...]) * dinv + b2_ref[...]
    h = jnp.maximum(z, 0.0)
    o = jnp.dot(h, w_ref[...], preferred_element_type=jnp.float32)
    o = o + bh_ref[...]
    t_ref[...] = o[:, 0:1]
    e_ref[...] = o[:, 1:2]

  grid = (N_NODES // BR,)
  return pl.pallas_call(
      body,
      grid=grid,
      in_specs=[
          pl.BlockSpec((1, BR, D_HID), lambda i: (0, i, 0)),
          pl.BlockSpec((1, BR, D_HID), lambda i: (1, i, 0)),
          pl.BlockSpec((BR, D_HID), lambda i: (i, 0)),
          pl.BlockSpec((BR, 1), lambda i: (i, 0)),
          pl.BlockSpec((1, D_HID), lambda i: (0, 0)),
          pl.BlockSpec((D_HID, 2), lambda i: (0, 0)),
          pl.BlockSpec((1, 2), lambda i: (0, 0)),
      ],
      out_specs=[
          pl.BlockSpec((BR, 1), lambda i: (i, 0)),
          pl.BlockSpec((BR, 1), lambda i: (i, 0)),
      ],
      out_shape=[
          jax.ShapeDtypeStruct((N_NODES, 1), jnp.float32),
          jax.ShapeDtypeStruct((N_NODES, 1), jnp.float32),
      ],
  )(agg, agg, scaled2, dinv, b2, W_he, b_he)


def kernel(x, edge_index, W1, b1, W2, b2, W_time, b_time, W_event, b_event):
  n_edges = edge_index.shape[1]
  assert n_edges % CHUNK == 0
  n_chunks = n_edges // CHUNK
  edges2d = edge_index.reshape(2, n_chunks, CHUNK)
  zeros64 = jnp.zeros((RPT, D_HID), jnp.float32)

  degp = _make_deg_count_kernel(n_chunks)(edges2d)
  deg = _deg_combine_kernel(degp)
  d = deg[:N_NODES].reshape(N_NODES, 1)

  scaled1, dinv = _tc_layer1(x, W1, d)

  edge_k = _make_edge_kernel(n_chunks)
  agg1 = edge_k(scaled1, edges2d, zeros64)
  scaled2 = _tc_layer2(agg1, scaled1, dinv, W2, b1.reshape(1, D_HID))

  agg2 = edge_k(scaled2, edges2d, zeros64)
  W_he = jnp.concatenate([W_time, W_event], axis=1)
  b_he = jnp.concatenate([b_time, b_event]).reshape(1, 2)
  t_out, e_out = _tc_heads(agg2, scaled2, dinv, b2.reshape(1, D_HID), W_he,
                           b_he)
  return (t_out, e_out)


# speedup vs baseline: 2.4356x; 1.0900x over previous
import functools

import jax
import jax.numpy as jnp
from jax import lax
from jax.experimental import pallas as pl
from jax.experimental.pallas import tpu as pltpu
from jax.experimental.pallas import tpu_sc as plsc

N_NODES = 10000
D_IN = 128
D_HID = 64

NC = 2
NS = 16
NW = NC * NS

CHUNK = 128
NP = 10112
RPT = NP // NS
NPD = 10240
DRT = NPD // NW

NBUF = 3

_sc_params = pltpu.CompilerParams(
    use_tc_tiling_on_sc=False, needs_layout_passes=False)
_mesh = plsc.VectorSubcoreMesh(
    core_axis_name="c", subcore_axis_name="s", num_cores=NC, num_subcores=NS)


def _chunk_split(n_chunks):
  base = n_chunks // NW
  left = n_chunks - base * NW
  return base, left


def _make_deg_count_kernel(n_chunks):
  base, left = _chunk_split(n_chunks)

  @functools.partial(
      pl.kernel,
      out_type=jax.ShapeDtypeStruct((NW, NPD), jnp.float32),
      mesh=_mesh,
      scratch_types=[
          pltpu.VMEM((base + 1, CHUNK), jnp.int32),
          pltpu.VMEM((NPD,), jnp.float32),
      ],
      compiler_params=_sc_params,
  )
  def deg_count(edges2d, out, idx_v, deg_v):
    cid = lax.axis_index("c")
    sid = lax.axis_index("s")
    wid = cid * NS + sid
    pltpu.sync_copy(edges2d.at[1].at[pl.ds(wid * base, base)],
                    idx_v.at[pl.ds(0, base)])

    @pl.when(wid < left)
    def _():
      pltpu.sync_copy(edges2d.at[1].at[pl.ds(NW * base + wid, 1)],
                      idx_v.at[pl.ds(base, 1)])

    zeros = jnp.zeros((16,), jnp.float32)

    @pl.loop(0, NPD // 16)
    def _(i):
      deg_v[pl.ds(i * 16, 16)] = zeros

    ones = jnp.ones((16,), jnp.float32)
    nmine = jnp.where(wid < left, base + 1, base)

    @pl.loop(0, nmine)
    def _(j):
      row = idx_v.at[j]
      for i in range(CHUNK // 16):
        idx16 = row[pl.ds(i * 16, 16)]
        plsc.addupdate_scatter(deg_v, [idx16], ones)

    pltpu.sync_copy(deg_v, out.at[wid])

  return deg_count


def _make_edge_kernel(n_chunks):
  base, left = _chunk_split(n_chunks)
  assert base % NBUF == 0 and base >= 2 * NBUF

  @functools.partial(
      pl.kernel,
      out_type=jax.ShapeDtypeStruct((NC, NP, D_HID), jnp.float32),
      mesh=_mesh,
      scratch_types=[
          pltpu.VMEM((base + 1, CHUNK), jnp.int32),
          pltpu.VMEM((base + 1, CHUNK), jnp.int32),
          pltpu.VMEM((NBUF, CHUNK, D_HID), jnp.float32),
          pltpu.VMEM((RPT, D_HID), jnp.float32),
          pltpu.VMEM_SHARED((NP, D_HID), jnp.float32),
      ] + [pltpu.SemaphoreType.DMA] * NBUF,
      compiler_params=_sc_params,
  )
  def edge_kernel(table, edges2d, out, src_v, dst_v, bufs,
                  stage_v, agg_sh, *gsem):
    cid = lax.axis_index("c")
    sid = lax.axis_index("s")
    wid = cid * NS + sid
    zeros = jnp.zeros((16,), jnp.float32)

    @pl.loop(0, RPT)
    def _(r):
      row = stage_v.at[r]
      for i in range(D_HID // 16):
        row[pl.ds(i * 16, 16)] = zeros

    pltpu.sync_copy(stage_v, agg_sh.at[pl.ds(sid * RPT, RPT)])
    pltpu.sync_copy(edges2d.at[0].at[pl.ds(wid * base, base)],
                    src_v.at[pl.ds(0, base)])
    pltpu.sync_copy(edges2d.at[1].at[pl.ds(wid * base, base)],
                    dst_v.at[pl.ds(0, base)])

    @pl.when(wid < left)
    def _():
      pltpu.sync_copy(edges2d.at[0].at[pl.ds(NW * base + wid, 1)],
                      src_v.at[pl.ds(base, 1)])
      pltpu.sync_copy(edges2d.at[1].at[pl.ds(NW * base + wid, 1)],
                      dst_v.at[pl.ds(base, 1)])

    plsc.subcore_barrier()

    for b in range(NBUF):
      pltpu.async_copy(table.at[src_v.at[b]], bufs.at[b], gsem[b])

    @pl.loop(0, base // NBUF)
    def _(g):
      bb = g * NBUF
      for b in range(NBUF):
        j = bb + b
        pltpu.make_async_copy(
            table.at[src_v.at[j]], bufs.at[b], gsem[b]).wait()
        pltpu.sync_copy(bufs.at[b], agg_sh.at[dst_v.at[j]], add=True)

        @pl.when(j + NBUF < base)
        def _():
          pltpu.async_copy(
              table.at[src_v.at[j + NBUF]], bufs.at[b], gsem[b])

    @pl.when(wid < left)
    def _():
      pltpu.async_copy(table.at[src_v.at[base]], bufs.at[0], gsem[0]).wait()
      pltpu.sync_copy(bufs.at[0], agg_sh.at[dst_v.at[base]], add=True)

    plsc.subcore_barrier()
    pltpu.sync_copy(agg_sh.at[pl.ds(sid * RPT, RPT)], stage_v)
    pltpu.sync_copy(stage_v, out.at[cid].at[pl.ds(sid * RPT, RPT)])

  return edge_kernel


BR = 2000


BR1 = 2048


def _tc_layer1(xp, W1, degp):
  def body(x_ref, w_ref, d_ref, scaled_ref, dinv_ref):
    deg_row = jnp.sum(d_ref[...], axis=0, keepdims=True)
    dinv = lax.rsqrt(jnp.transpose(deg_row) + 1.0)
    h = jnp.dot(x_ref[...], w_ref[...], preferred_element_type=jnp.float32)
    scaled_ref[...] = h * dinv
    dinv_ref[...] = dinv

  grid = (NPD // BR1,)
  return pl.pallas_call(
      body,
      grid=grid,
      in_specs=[
          pl.BlockSpec((BR1, D_IN), lambda i: (i, 0)),
          pl.BlockSpec((D_IN, D_HID), lambda i: (0, 0)),
          pl.BlockSpec((NW, BR1), lambda i: (0, i)),
      ],
      out_specs=[
          pl.BlockSpec((BR1, D_HID), lambda i: (i, 0)),
          pl.BlockSpec((BR1, 1), lambda i: (i, 0)),
      ],
      out_shape=[
          jax.ShapeDtypeStruct((NPD, D_HID), jnp.float32),
          jax.ShapeDtypeStruct((NPD, 1), jnp.float32),
      ],
  )(xp, W1, degp)


def _tc_layer2(agg, scaled1, dinv, W2, b1):
  def body(a0_ref, a1_ref, s_ref, dinv_ref, w_ref, b_ref, out_ref):
    dinv = dinv_ref[...]
    z = (a0_ref[0] + a1_ref[0] + s_ref[...]) * dinv + b_ref[...]
    h1 = jnp.maximum(z, 0.0)
    out_ref[...] = jnp.dot(
        h1, w_ref[...], preferred_element_type=jnp.float32) * dinv

  grid = (N_NODES // BR,)
  return pl.pallas_call(
      body,
      grid=grid,
      in_specs=[
          pl.BlockSpec((1, BR, D_HID), lambda i: (0, i, 0)),
          pl.BlockSpec((1, BR, D_HID), lambda i: (1, i, 0)),
          pl.BlockSpec((BR, D_HID), lambda i: (i, 0)),
          pl.BlockSpec((BR, 1), lambda i: (i, 0)),
          pl.BlockSpec((D_HID, D_HID), lambda i: (0, 0)),
          pl.BlockSpec((1, D_HID), lambda i: (0, 0)),
      ],
      out_specs=pl.BlockSpec((BR, D_HID), lambda i: (i, 0)),
      out_shape=jax.ShapeDtypeStruct((N_NODES, D_HID), jnp.float32),
  )(agg, agg, scaled1, dinv, W2, b1)


def _tc_heads(agg, scaled2, dinv, b2, W_he, b_he):
  def body(a0_ref, a1_ref, s_ref, dinv_ref, b2_ref, w_ref, bh_ref,
           t_ref, e_ref):
    dinv = dinv_ref[...]
    z = (a0_ref[0] + a1_ref[0] + s_ref[...]) * dinv + b2_ref[...]
    h = jnp.maximum(z, 0.0)
    o = jnp.dot(h, w_ref[...], preferred_element_type=jnp.float32)
    o = o + bh_ref[...]
    t_ref[...] = o[:, 0:1]
    e_ref[...] = o[:, 1:2]

  grid = (N_NODES // BR,)
  return pl.pallas_call(
      body,
      grid=grid,
      in_specs=[
          pl.BlockSpec((1, BR, D_HID), lambda i: (0, i, 0)),
          pl.BlockSpec((1, BR, D_HID), lambda i: (1, i, 0)),
          pl.BlockSpec((BR, D_HID), lambda i: (i, 0)),
          pl.BlockSpec((BR, 1), lambda i: (i, 0)),
          pl.BlockSpec((1, D_HID), lambda i: (0, 0)),
          pl.BlockSpec((D_HID, 2), lambda i: (0, 0)),
          pl.BlockSpec((1, 2), lambda i: (0, 0)),
      ],
      out_specs=[
          pl.BlockSpec((BR, 1), lambda i: (i, 0)),
          pl.BlockSpec((BR, 1), lambda i: (i, 0)),
      ],
      out_shape=[
          jax.ShapeDtypeStruct((N_NODES, 1), jnp.float32),
          jax.ShapeDtypeStruct((N_NODES, 1), jnp.float32),
      ],
  )(agg, agg, scaled2, dinv, b2, W_he, b_he)


def kernel(x, edge_index, W1, b1, W2, b2, W_time, b_time, W_event, b_event):
  n_edges = edge_index.shape[1]
  assert n_edges % CHUNK == 0
  n_chunks = n_edges // CHUNK
  edges2d = edge_index.reshape(2, n_chunks, CHUNK)

  degp = _make_deg_count_kernel(n_chunks)(edges2d)
  xp = jnp.concatenate(
      [x, jnp.zeros((NPD - N_NODES, D_IN), jnp.float32)])
  scaled1, dinv = _tc_layer1(xp, W1, degp)

  edge_k = _make_edge_kernel(n_chunks)
  agg1 = edge_k(scaled1, edges2d)
  scaled2 = _tc_layer2(agg1, scaled1, dinv, W2, b1.reshape(1, D_HID))

  agg2 = edge_k(scaled2, edges2d)
  W_he = jnp.concatenate([W_time, W_event], axis=1)
  b_he = jnp.concatenate([b_time, b_event]).reshape(1, 2)
  t_out, e_out = _tc_heads(agg2, scaled2, dinv, b2.reshape(1, D_HID), W_he,
                           b_he)
  return (t_out, e_out)


# speedup vs baseline: 2.4864x; 1.0209x over previous
import functools

import jax
import jax.numpy as jnp
from jax import lax
from jax.experimental import pallas as pl
from jax.experimental.pallas import tpu as pltpu
from jax.experimental.pallas import tpu_sc as plsc

N_NODES = 10000
D_IN = 128
D_HID = 64

NC = 2
NS = 16
NW = NC * NS

CHUNK = 128
NP = 10112
RPT = NP // NS
NPD = 10240
DRT = NPD // NW

NBUF = 3

_sc_params = pltpu.CompilerParams(
    use_tc_tiling_on_sc=False, needs_layout_passes=False)
_mesh = plsc.VectorSubcoreMesh(
    core_axis_name="c", subcore_axis_name="s", num_cores=NC, num_subcores=NS)


def _chunk_split(n_chunks):
  base = n_chunks // NW
  left = n_chunks - base * NW
  return base, left


def _make_deg_count_kernel(n_chunks):
  base, left = _chunk_split(n_chunks)

  @functools.partial(
      pl.kernel,
      out_type=jax.ShapeDtypeStruct((NW, NPD), jnp.float32),
      mesh=_mesh,
      scratch_types=[
          pltpu.VMEM((base + 1, CHUNK), jnp.int32),
          pltpu.VMEM((NPD,), jnp.float32),
      ],
      compiler_params=_sc_params,
  )
  def deg_count(edges2d, out, idx_v, deg_v):
    cid = lax.axis_index("c")
    sid = lax.axis_index("s")
    wid = cid * NS + sid
    pltpu.sync_copy(edges2d.at[1].at[pl.ds(wid * base, base)],
                    idx_v.at[pl.ds(0, base)])

    @pl.when(wid < left)
    def _():
      pltpu.sync_copy(edges2d.at[1].at[pl.ds(NW * base + wid, 1)],
                      idx_v.at[pl.ds(base, 1)])

    zeros = jnp.zeros((16,), jnp.float32)

    @pl.loop(0, NPD // 16)
    def _(i):
      deg_v[pl.ds(i * 16, 16)] = zeros

    ones = jnp.ones((16,), jnp.float32)
    nmine = jnp.where(wid < left, base + 1, base)

    @pl.loop(0, nmine)
    def _(j):
      row = idx_v.at[j]
      for i in range(CHUNK // 16):
        idx16 = row[pl.ds(i * 16, 16)]
        plsc.addupdate_scatter(deg_v, [idx16], ones)

    pltpu.sync_copy(deg_v, out.at[wid])

  return deg_count


def _make_edge_kernel(n_chunks):
  base, left = _chunk_split(n_chunks)
  assert base % NBUF == 0 and base >= 2 * NBUF

  @functools.partial(
      pl.kernel,
      out_type=jax.ShapeDtypeStruct((NC, NP, D_HID), jnp.float32),
      mesh=_mesh,
      scratch_types=[
          pltpu.VMEM((base + 1, CHUNK), jnp.int32),
          pltpu.VMEM((base + 1, CHUNK), jnp.int32),
          pltpu.VMEM((NBUF, CHUNK, D_HID), jnp.float32),
          pltpu.VMEM((RPT, D_HID), jnp.float32),
          pltpu.VMEM_SHARED((NP, D_HID), jnp.float32),
      ] + [pltpu.SemaphoreType.DMA] * NBUF,
      compiler_params=_sc_params,
  )
  def edge_kernel(table, edges2d, out, src_v, dst_v, bufs,
                  stage_v, agg_sh, *gsem):
    cid = lax.axis_index("c")
    sid = lax.axis_index("s")
    wid = cid * NS + sid
    ld_src = pltpu.async_copy(edges2d.at[0].at[pl.ds(wid * base, base)],
                              src_v.at[pl.ds(0, base)], gsem[0])
    ld_dst = pltpu.async_copy(edges2d.at[1].at[pl.ds(wid * base, base)],
                              dst_v.at[pl.ds(0, base)], gsem[1])
    zeros = jnp.zeros((16,), jnp.float32)

    @pl.loop(0, RPT)
    def _(r):
      row = stage_v.at[r]
      for i in range(D_HID // 16):
        row[pl.ds(i * 16, 16)] = zeros

    pltpu.sync_copy(stage_v, agg_sh.at[pl.ds(sid * RPT, RPT)])
    ld_src.wait()
    ld_dst.wait()

    @pl.when(wid < left)
    def _():
      pltpu.sync_copy(edges2d.at[0].at[pl.ds(NW * base + wid, 1)],
                      src_v.at[pl.ds(base, 1)])
      pltpu.sync_copy(edges2d.at[1].at[pl.ds(NW * base + wid, 1)],
                      dst_v.at[pl.ds(base, 1)])

    plsc.subcore_barrier()

    for b in range(NBUF):
      pltpu.async_copy(table.at[src_v.at[b]], bufs.at[b], gsem[b])

    @pl.loop(0, base // NBUF)
    def _(g):
      bb = g * NBUF
      for b in range(NBUF):
        j = bb + b
        pltpu.make_async_copy(
            table.at[src_v.at[j]], bufs.at[b], gsem[b]).wait()
        pltpu.sync_copy(bufs.at[b], agg_sh.at[dst_v.at[j]], add=True)

        @pl.when(j + NBUF < base)
        def _():
          pltpu.async_copy(
              table.at[src_v.at[j + NBUF]], bufs.at[b], gsem[b])

    @pl.when(wid < left)
    def _():
      pltpu.async_copy(table.at[src_v.at[base]], bufs.at[0], gsem[0]).wait()
      pltpu.sync_copy(bufs.at[0], agg_sh.at[dst_v.at[base]], add=True)

    plsc.subcore_barrier()
    pltpu.sync_copy(agg_sh.at[pl.ds(sid * RPT, RPT)], stage_v)
    pltpu.sync_copy(stage_v, out.at[cid].at[pl.ds(sid * RPT, RPT)])

  return edge_kernel


BR = 2000


BR1 = 2048


def _tc_layer1(xp, W1, degp):
  def body(x_ref, w_ref, d_ref, scaled_ref, dinv_ref):
    deg_row = jnp.sum(d_ref[...], axis=0, keepdims=True)
    dinv = lax.rsqrt(jnp.transpose(deg_row) + 1.0)
    h = jnp.dot(x_ref[...], w_ref[...], preferred_element_type=jnp.float32)
    scaled_ref[...] = h * dinv
    dinv_ref[...] = dinv

  grid = (NPD // BR1,)
  return pl.pallas_call(
      body,
      grid=grid,
      in_specs=[
          pl.BlockSpec((BR1, D_IN), lambda i: (i, 0)),
          pl.BlockSpec((D_IN, D_HID), lambda i: (0, 0)),
          pl.BlockSpec((NW, BR1), lambda i: (0, i)),
      ],
      out_specs=[
          pl.BlockSpec((BR1, D_HID), lambda i: (i, 0)),
          pl.BlockSpec((BR1, 1), lambda i: (i, 0)),
      ],
      out_shape=[
          jax.ShapeDtypeStruct((NPD, D_HID), jnp.float32),
          jax.ShapeDtypeStruct((NPD, 1), jnp.float32),
      ],
  )(xp, W1, degp)


def _tc_layer2(agg, scaled1, dinv, W2, b1):
  def body(a0_ref, a1_ref, s_ref, dinv_ref, w_ref, b_ref, out_ref):
    dinv = dinv_ref[...]
    z = (a0_ref[0] + a1_ref[0] + s_ref[...]) * dinv + b_ref[...]
    h1 = jnp.maximum(z, 0.0)
    out_ref[...] = jnp.dot(
        h1, w_ref[...], preferred_element_type=jnp.float32) * dinv

  grid = (N_NODES // BR,)
  return pl.pallas_call(
      body,
      grid=grid,
      in_specs=[
          pl.BlockSpec((1, BR, D_HID), lambda i: (0, i, 0)),
          pl.BlockSpec((1, BR, D_HID), lambda i: (1, i, 0)),
          pl.BlockSpec((BR, D_HID), lambda i: (i, 0)),
          pl.BlockSpec((BR, 1), lambda i: (i, 0)),
          pl.BlockSpec((D_HID, D_HID), lambda i: (0, 0)),
          pl.BlockSpec((1, D_HID), lambda i: (0, 0)),
      ],
      out_specs=pl.BlockSpec((BR, D_HID), lambda i: (i, 0)),
      out_shape=jax.ShapeDtypeStruct((N_NODES, D_HID), jnp.float32),
  )(agg, agg, scaled1, dinv, W2, b1)


def _tc_heads(agg, scaled2, dinv, b2, W_he, b_he):
  def body(a0_ref, a1_ref, s_ref, dinv_ref, b2_ref, w_ref, bh_ref,
           t_ref, e_ref):
    dinv = dinv_ref[...]
    z = (a0_ref[0] + a1_ref[0] + s_ref[...]) * dinv + b2_ref[...]
    h = jnp.maximum(z, 0.0)
    o = jnp.dot(h, w_ref[...], preferred_element_type=jnp.float32)
    o = o + bh_ref[...]
    t_ref[...] = o[:, 0:1]
    e_ref[...] = o[:, 1:2]

  grid = (N_NODES // BR,)
  return pl.pallas_call(
      body,
      grid=grid,
      in_specs=[
          pl.BlockSpec((1, BR, D_HID), lambda i: (0, i, 0)),
          pl.BlockSpec((1, BR, D_HID), lambda i: (1, i, 0)),
          pl.BlockSpec((BR, D_HID), lambda i: (i, 0)),
          pl.BlockSpec((BR, 1), lambda i: (i, 0)),
          pl.BlockSpec((1, D_HID), lambda i: (0, 0)),
          pl.BlockSpec((D_HID, 2), lambda i: (0, 0)),
          pl.BlockSpec((1, 2), lambda i: (0, 0)),
      ],
      out_specs=[
          pl.BlockSpec((BR, 1), lambda i: (i, 0)),
          pl.BlockSpec((BR, 1), lambda i: (i, 0)),
      ],
      out_shape=[
          jax.ShapeDtypeStruct((N_NODES, 1), jnp.float32),
          jax.ShapeDtypeStruct((N_NODES, 1), jnp.float32),
      ],
  )(agg, agg, scaled2, dinv, b2, W_he, b_he)


def kernel(x, edge_index, W1, b1, W2, b2, W_time, b_time, W_event, b_event):
  n_edges = edge_index.shape[1]
  assert n_edges % CHUNK == 0
  n_chunks = n_edges // CHUNK
  edges2d = edge_index.reshape(2, n_chunks, CHUNK)

  degp = _make_deg_count_kernel(n_chunks)(edges2d)
  xp = jnp.concatenate(
      [x, jnp.zeros((NPD - N_NODES, D_IN), jnp.float32)])
  scaled1, dinv = _tc_layer1(xp, W1, degp)

  edge_k = _make_edge_kernel(n_chunks)
  agg1 = edge_k(scaled1, edges2d)
  scaled2 = _tc_layer2(agg1, scaled1, dinv, W2, b1.reshape(1, D_HID))

  agg2 = edge_k(scaled2, edges2d)
  W_he = jnp.concatenate([W_time, W_event], axis=1)
  b_he = jnp.concatenate([b_time, b_event]).reshape(1, 2)
  t_out, e_out = _tc_heads(agg2, scaled2, dinv, b2.reshape(1, D_HID), W_he,
                           b_he)
  return (t_out, e_out)


# speedup vs baseline: 2.5040x; 1.0071x over previous
import functools

import jax
import jax.numpy as jnp
from jax import lax
from jax.experimental import pallas as pl
from jax.experimental.pallas import tpu as pltpu
from jax.experimental.pallas import tpu_sc as plsc

N_NODES = 10000
D_IN = 128
D_HID = 64

NC = 2
NS = 16
NW = NC * NS

CHUNK = 128
NP = 10112
RPT = NP // NS
NPD = 10240
DRT = NPD // NW

NBUF = 3

_sc_params = pltpu.CompilerParams(
    use_tc_tiling_on_sc=False, needs_layout_passes=False)
_mesh = plsc.VectorSubcoreMesh(
    core_axis_name="c", subcore_axis_name="s", num_cores=NC, num_subcores=NS)


def _chunk_split(n_chunks):
  base = n_chunks // NW
  left = n_chunks - base * NW
  return base, left


def _make_deg_count_kernel(n_chunks):
  base, left = _chunk_split(n_chunks)

  @functools.partial(
      pl.kernel,
      out_type=jax.ShapeDtypeStruct((NW, NPD), jnp.float32),
      mesh=_mesh,
      scratch_types=[
          pltpu.VMEM((base + 1, CHUNK), jnp.int32),
          pltpu.VMEM((NPD,), jnp.float32),
          pltpu.SemaphoreType.DMA,
      ],
      compiler_params=_sc_params,
  )
  def deg_count(edges2d, out, idx_v, deg_v, sem):
    cid = lax.axis_index("c")
    sid = lax.axis_index("s")
    wid = cid * NS + sid
    ld = pltpu.async_copy(edges2d.at[1].at[pl.ds(wid * base, base)],
                          idx_v.at[pl.ds(0, base)], sem)
    zeros = jnp.zeros((16,), jnp.float32)

    @pl.loop(0, NPD // 16)
    def _(i):
      deg_v[pl.ds(i * 16, 16)] = zeros

    ld.wait()

    @pl.when(wid < left)
    def _():
      pltpu.sync_copy(edges2d.at[1].at[pl.ds(NW * base + wid, 1)],
                      idx_v.at[pl.ds(base, 1)])

    ones = jnp.ones((16,), jnp.float32)
    nmine = jnp.where(wid < left, base + 1, base)

    @pl.loop(0, nmine)
    def _(j):
      row = idx_v.at[j]
      for i in range(CHUNK // 16):
        idx16 = row[pl.ds(i * 16, 16)]
        plsc.addupdate_scatter(deg_v, [idx16], ones)

    pltpu.sync_copy(deg_v, out.at[wid])

  return deg_count


def _make_edge_kernel(n_chunks):
  base, left = _chunk_split(n_chunks)
  assert base % NBUF == 0 and base >= 2 * NBUF

  @functools.partial(
      pl.kernel,
      out_type=jax.ShapeDtypeStruct((NC, NP, D_HID), jnp.float32),
      mesh=_mesh,
      scratch_types=[
          pltpu.VMEM((base + 1, CHUNK), jnp.int32),
          pltpu.VMEM((base + 1, CHUNK), jnp.int32),
          pltpu.VMEM((NBUF, CHUNK, D_HID), jnp.float32),
          pltpu.VMEM((RPT, D_HID), jnp.float32),
          pltpu.VMEM_SHARED((NP, D_HID), jnp.float32),
      ] + [pltpu.SemaphoreType.DMA] * NBUF,
      compiler_params=_sc_params,
  )
  def edge_kernel(table, edges2d, out, src_v, dst_v, bufs,
                  stage_v, agg_sh, *gsem):
    cid = lax.axis_index("c")
    sid = lax.axis_index("s")
    wid = cid * NS + sid
    ld_src = pltpu.async_copy(edges2d.at[0].at[pl.ds(wid * base, base)],
                              src_v.at[pl.ds(0, base)], gsem[0])
    ld_dst = pltpu.async_copy(edges2d.at[1].at[pl.ds(wid * base, base)],
                              dst_v.at[pl.ds(0, base)], gsem[1])
    zeros = jnp.zeros((16,), jnp.float32)

    @pl.loop(0, RPT)
    def _(r):
      row = stage_v.at[r]
      for i in range(D_HID // 16):
        row[pl.ds(i * 16, 16)] = zeros

    pltpu.sync_copy(stage_v, agg_sh.at[pl.ds(sid * RPT, RPT)])
    ld_src.wait()
    ld_dst.wait()

    @pl.when(wid < left)
    def _():
      pltpu.sync_copy(edges2d.at[0].at[pl.ds(NW * base + wid, 1)],
                      src_v.at[pl.ds(base, 1)])
      pltpu.sync_copy(edges2d.at[1].at[pl.ds(NW * base + wid, 1)],
                      dst_v.at[pl.ds(base, 1)])

    plsc.subcore_barrier()

    for b in range(NBUF):
      pltpu.async_copy(table.at[src_v.at[b]], bufs.at[b], gsem[b])

    @pl.loop(0, base // NBUF)
    def _(g):
      bb = g * NBUF
      for b in range(NBUF):
        j = bb + b
        pltpu.make_async_copy(
            table.at[src_v.at[j]], bufs.at[b], gsem[b]).wait()
        pltpu.sync_copy(bufs.at[b], agg_sh.at[dst_v.at[j]], add=True)

        @pl.when(j + NBUF < base)
        def _():
          pltpu.async_copy(
              table.at[src_v.at[j + NBUF]], bufs.at[b], gsem[b])

    @pl.when(wid < left)
    def _():
      pltpu.async_copy(table.at[src_v.at[base]], bufs.at[0], gsem[0]).wait()
      pltpu.sync_copy(bufs.at[0], agg_sh.at[dst_v.at[base]], add=True)

    plsc.subcore_barrier()
    pltpu.sync_copy(agg_sh.at[pl.ds(sid * RPT, RPT)], stage_v)
    pltpu.sync_copy(stage_v, out.at[cid].at[pl.ds(sid * RPT, RPT)])

  return edge_kernel


BR = 2000


BR1 = 2048


def _tc_layer1(xp, W1, degp):
  def body(x_ref, w_ref, d_ref, scaled_ref, dinv_ref):
    deg_row = jnp.sum(d_ref[...], axis=0, keepdims=True)
    dinv = lax.rsqrt(jnp.transpose(deg_row) + 1.0)
    h = jnp.dot(x_ref[...], w_ref[...], preferred_element_type=jnp.float32)
    scaled_ref[...] = h * dinv
    dinv_ref[...] = dinv

  grid = (NPD // BR1,)
  return pl.pallas_call(
      body,
      grid=grid,
      in_specs=[
          pl.BlockSpec((BR1, D_IN), lambda i: (i, 0)),
          pl.BlockSpec((D_IN, D_HID), lambda i: (0, 0)),
          pl.BlockSpec((NW, BR1), lambda i: (0, i)),
      ],
      out_specs=[
          pl.BlockSpec((BR1, D_HID), lambda i: (i, 0)),
          pl.BlockSpec((BR1, 1), lambda i: (i, 0)),
      ],
      out_shape=[
          jax.ShapeDtypeStruct((NPD, D_HID), jnp.float32),
          jax.ShapeDtypeStruct((NPD, 1), jnp.float32),
      ],
  )(xp, W1, degp)


def _tc_layer2(agg, scaled1, dinv, W2, b1):
  def body(a0_ref, a1_ref, s_ref, dinv_ref, w_ref, b_ref, out_ref):
    dinv = dinv_ref[...]
    z = (a0_ref[0] + a1_ref[0] + s_ref[...]) * dinv + b_ref[...]
    h1 = jnp.maximum(z, 0.0)
    out_ref[...] = jnp.dot(
        h1, w_ref[...], preferred_element_type=jnp.float32) * dinv

  grid = (N_NODES // BR,)
  return pl.pallas_call(
      body,
      grid=grid,
      in_specs=[
          pl.BlockSpec((1, BR, D_HID), lambda i: (0, i, 0)),
          pl.BlockSpec((1, BR, D_HID), lambda i: (1, i, 0)),
          pl.BlockSpec((BR, D_HID), lambda i: (i, 0)),
          pl.BlockSpec((BR, 1), lambda i: (i, 0)),
          pl.BlockSpec((D_HID, D_HID), lambda i: (0, 0)),
          pl.BlockSpec((1, D_HID), lambda i: (0, 0)),
      ],
      out_specs=pl.BlockSpec((BR, D_HID), lambda i: (i, 0)),
      out_shape=jax.ShapeDtypeStruct((N_NODES, D_HID), jnp.float32),
  )(agg, agg, scaled1, dinv, W2, b1)


def _tc_heads(agg, scaled2, dinv, b2, W_he, b_he):
  def body(a0_ref, a1_ref, s_ref, dinv_ref, b2_ref, w_ref, bh_ref,
           t_ref, e_ref):
    dinv = dinv_ref[...]
    z = (a0_ref[0] + a1_ref[0] + s_ref[...]) * dinv + b2_ref[...]
    h = jnp.maximum(z, 0.0)
    o = jnp.dot(h, w_ref[...], preferred_element_type=jnp.float32)
    o = o + bh_ref[...]
    t_ref[...] = o[:, 0:1]
    e_ref[...] = o[:, 1:2]

  grid = (N_NODES // BR,)
  return pl.pallas_call(
      body,
      grid=grid,
      in_specs=[
          pl.BlockSpec((1, BR, D_HID), lambda i: (0, i, 0)),
          pl.BlockSpec((1, BR, D_HID), lambda i: (1, i, 0)),
          pl.BlockSpec((BR, D_HID), lambda i: (i, 0)),
          pl.BlockSpec((BR, 1), lambda i: (i, 0)),
          pl.BlockSpec((1, D_HID), lambda i: (0, 0)),
          pl.BlockSpec((D_HID, 2), lambda i: (0, 0)),
          pl.BlockSpec((1, 2), lambda i: (0, 0)),
      ],
      out_specs=[
          pl.BlockSpec((BR, 1), lambda i: (i, 0)),
          pl.BlockSpec((BR, 1), lambda i: (i, 0)),
      ],
      out_shape=[
          jax.ShapeDtypeStruct((N_NODES, 1), jnp.float32),
          jax.ShapeDtypeStruct((N_NODES, 1), jnp.float32),
      ],
  )(agg, agg, scaled2, dinv, b2, W_he, b_he)


def kernel(x, edge_index, W1, b1, W2, b2, W_time, b_time, W_event, b_event):
  n_edges = edge_index.shape[1]
  assert n_edges % CHUNK == 0
  n_chunks = n_edges // CHUNK
  edges2d = edge_index.reshape(2, n_chunks, CHUNK)

  degp = _make_deg_count_kernel(n_chunks)(edges2d)
  xp = jnp.concatenate(
      [x, jnp.zeros((NPD - N_NODES, D_IN), jnp.float32)])
  scaled1, dinv = _tc_layer1(xp, W1, degp)

  edge_k = _make_edge_kernel(n_chunks)
  agg1 = edge_k(scaled1, edges2d)
  scaled2 = _tc_layer2(agg1, scaled1, dinv, W2, b1.reshape(1, D_HID))

  agg2 = edge_k(scaled2, edges2d)
  W_he = jnp.concatenate([W_time, W_event], axis=1)
  b_he = jnp.concatenate([b_time, b_event]).reshape(1, 2)
  t_out, e_out = _tc_heads(agg2, scaled2, dinv, b2.reshape(1, D_HID), W_he,
                           b_he)
  return (t_out, e_out)
